# baseline (speedup 1.0000x reference)
import functools

import jax
import jax.numpy as jnp
from jax import lax
from jax.experimental import pallas as pl
from jax.experimental.pallas import tpu as pltpu
from jax.experimental.pallas import tpu_sc as plsc

ENT_N = 1000000
REL_N = 1000000
D = 64
BS = 4096
BSEQ = 8192
REG_C = 1e-05

NW = 32
E_IDX = 2 * BSEQ
E_PER = E_IDX // NW
R_PER = BSEQ // NW


@functools.cache
def _sc_gather_fn():
    mesh = plsc.VectorSubcoreMesh(core_axis_name="c", subcore_axis_name="s")

    @functools.partial(
        pl.kernel,
        mesh=mesh,
        out_type=[
            jax.ShapeDtypeStruct((D, E_IDX), jnp.float32),
            jax.ShapeDtypeStruct((D, E_IDX), jnp.float32),
            jax.ShapeDtypeStruct((D, BSEQ), jnp.float32),
            jax.ShapeDtypeStruct((D, BSEQ), jnp.float32),
        ],
        scratch_types=[
            pltpu.VMEM((D * E_PER,), jnp.int32),
            pltpu.VMEM((D * R_PER,), jnp.int32),
            pltpu.VMEM((E_PER,), jnp.float32),
            pltpu.VMEM((E_PER,), jnp.float32),
            pltpu.VMEM((R_PER,), jnp.float32),
            pltpu.VMEM((R_PER,), jnp.float32),
            pltpu.SemaphoreType.DMA,
        ],
        compiler_params=pltpu.CompilerParams(use_tc_tiling_on_sc=False),
    )
    def _sc_gather(iw_e, iw_r, ent_f, etr_f, rel_f, rtr_f,
                   ht_o, httr_o, r_o, rtr_o,
                   ie_v, ir_v, be1, be2, br1, br2, sem):
        wid = lax.axis_index("s") * 2 + lax.axis_index("c")
        be = wid * E_PER
        br = wid * R_PER
        pltpu.sync_copy(iw_e.at[wid], ie_v)
        pltpu.sync_copy(iw_r.at[wid], ir_v)

        def body(d, _):
            se = pl.ds(d * E_PER, E_PER)
            sr = pl.ds(d * R_PER, R_PER)
            c1 = pltpu.async_copy(ent_f.at[ie_v.at[se]], be1, sem)
            c2 = pltpu.async_copy(etr_f.at[ie_v.at[se]], be2, sem)
            c3 = pltpu.async_copy(rel_f.at[ir_v.at[sr]], br1, sem)
            c4 = pltpu.async_copy(rtr_f.at[ir_v.at[sr]], br2, sem)
            c1.wait()
            c2.wait()
            c3.wait()
            c4.wait()
            pltpu.sync_copy(be1, ht_o.at[d, pl.ds(be, E_PER)])
            pltpu.sync_copy(be2, httr_o.at[d, pl.ds(be, E_PER)])
            pltpu.sync_copy(br1, r_o.at[d, pl.ds(br, R_PER)])
            pltpu.sync_copy(br2, rtr_o.at[d, pl.ds(br, R_PER)])
            return 0

        lax.fori_loop(0, D, body, 0)

    return _sc_gather


def _loss_body(htT_ref, httrT_ref, rT_ref, rtrT_ref, out_ref):
    r_tr = rtrT_ref[...]

    def transfer(e, etr):
        dot = jnp.sum(e * etr, axis=0, keepdims=True)
        e2 = e + dot * r_tr
        n = jnp.sqrt(jnp.sum(e2 * e2, axis=0, keepdims=True))
        return e2 / jnp.maximum(n, 1e-12)

    h = transfer(htT_ref[:, 0:BSEQ], httrT_ref[:, 0:BSEQ])
    t = transfer(htT_ref[:, BSEQ:E_IDX], httrT_ref[:, BSEQ:E_IDX])
    a = jnp.abs(h + rT_ref[...] - t + 1e-06)
    diff = a[:, 0:BS] - a[:, BS:BSEQ]
    rows = jnp.sum(diff, axis=0, keepdims=True)
    out_ref[0, 0] = jnp.sum(jnp.maximum(rows + 1.0, 0.0)) * (1.0 / BS)


_loss_call = pl.pallas_call(
    _loss_body,
    out_specs=pl.BlockSpec(memory_space=pltpu.SMEM),
    out_shape=jax.ShapeDtypeStruct((1, 1), jnp.float32),
)

NCHUNK = 8192
NGRID = -(-ENT_N // NCHUNK)


def _norm_body(a_ref, b_ref, c_ref, d_ref, out_ref):
    i = pl.program_id(0)
    rem = ENT_N - i * NCHUNK

    @pl.when(i == 0)
    def _():
        out_ref[0, 0] = 0.0

    @pl.when(rem >= NCHUNK)
    def _():
        s_ent = jnp.sum(jnp.abs(a_ref[...])) + jnp.sum(jnp.abs(c_ref[...]))
        s_rel = jnp.sum(jnp.abs(b_ref[...])) + jnp.sum(jnp.abs(d_ref[...]))
        out_ref[0, 0] += s_ent * (1.0 / ENT_N) + s_rel * (1.0 / REL_N)

    @pl.when(rem < NCHUNK)
    def _():
        m = jax.lax.broadcasted_iota(jnp.int32, (D, NCHUNK), 1) < rem

        def masked(ref):
            return jnp.sum(jnp.where(m, jnp.abs(ref[...]), 0.0))

        s_ent = masked(a_ref) + masked(c_ref)
        s_rel = masked(b_ref) + masked(d_ref)
        out_ref[0, 0] += s_ent * (1.0 / ENT_N) + s_rel * (1.0 / REL_N)


_norm_call = pl.pallas_call(
    _norm_body,
    grid=(NGRID,),
    in_specs=[pl.BlockSpec((D, NCHUNK), lambda i: (0, i))] * 4,
    out_specs=pl.BlockSpec(memory_space=pltpu.SMEM),
    out_shape=jax.ShapeDtypeStruct((1, 1), jnp.float32),
)


def kernel(input, ent_emb, rel_emb, ent_transfer, rel_transfer):
    idx_e = jnp.concatenate([input[:, 0], input[:, 2]])
    idx_r = input[:, 1]
    doff = jnp.arange(D, dtype=jnp.int32) * ENT_N
    iw_e = (idx_e.reshape(NW, 1, E_PER) + doff.reshape(1, D, 1)).reshape(NW, D * E_PER)
    iw_r = (idx_r.reshape(NW, 1, R_PER) + doff.reshape(1, D, 1)).reshape(NW, D * R_PER)
    entT = ent_emb.T
    etrT = ent_transfer.T
    relT = rel_emb.T
    rtrT = rel_transfer.T
    htT, httrT, rT, rtrT_g = _sc_gather_fn()(
        iw_e, iw_r, entT.reshape(-1), etrT.reshape(-1),
        relT.reshape(-1), rtrT.reshape(-1))
    loss = _loss_call(htT, httrT, rT, rtrT_g)
    norm = _norm_call(entT, relT, etrT, rtrT)
    return loss[0, 0] + norm[0, 0] * REG_C


# speedup vs baseline: 8.4816x; 8.4816x over previous
import functools

import jax
import jax.numpy as jnp
from jax import lax
from jax.experimental import pallas as pl
from jax.experimental.pallas import tpu as pltpu
from jax.experimental.pallas import tpu_sc as plsc

ENT_N = 1000000
REL_N = 1000000
D = 64
BS = 4096
BSEQ = 8192
REG_C = 1e-05

NW = 32
E_IDX = 2 * BSEQ
E_PER = E_IDX // NW
R_PER = BSEQ // NW


GCHUNK = 128


@functools.cache
def _sc_gather_fn():
    mesh = plsc.VectorSubcoreMesh(core_axis_name="c", subcore_axis_name="s")

    @functools.partial(
        pl.kernel,
        mesh=mesh,
        out_type=[
            jax.ShapeDtypeStruct((E_IDX, D), jnp.float32),
            jax.ShapeDtypeStruct((E_IDX, D), jnp.float32),
            jax.ShapeDtypeStruct((BSEQ, D), jnp.float32),
            jax.ShapeDtypeStruct((BSEQ, D), jnp.float32),
        ],
        scratch_types=[
            pltpu.VMEM((E_PER,), jnp.int32),
            pltpu.VMEM((R_PER,), jnp.int32),
            pltpu.VMEM((E_PER, D), jnp.float32),
            pltpu.VMEM((E_PER, D), jnp.float32),
            pltpu.VMEM((R_PER, D), jnp.float32),
            pltpu.VMEM((R_PER, D), jnp.float32),
            pltpu.SemaphoreType.DMA,
        ],
        compiler_params=pltpu.CompilerParams(use_tc_tiling_on_sc=False),
    )
    def _sc_gather(idx_e_hbm, idx_r_hbm, ent_emb, ent_tr, rel_emb, rel_tr,
                   ht_out, httr_out, r_out, rtr_out,
                   idx_e_v, idx_r_v, rows_he, rows_htr, rows_r, rows_rtr, sem):
        wid = lax.axis_index("s") * 2 + lax.axis_index("c")
        be = wid * E_PER
        br = wid * R_PER
        pltpu.sync_copy(idx_e_hbm.at[pl.ds(be, E_PER)], idx_e_v)
        pltpu.sync_copy(idx_r_hbm.at[pl.ds(br, R_PER)], idx_r_v)
        copies = []
        for j in range(E_PER // GCHUNK):
            s = pl.ds(j * GCHUNK, GCHUNK)
            copies.append(pltpu.async_copy(ent_emb.at[idx_e_v.at[s]], rows_he.at[s], sem))
            copies.append(pltpu.async_copy(ent_tr.at[idx_e_v.at[s]], rows_htr.at[s], sem))
        for j in range(R_PER // GCHUNK):
            s = pl.ds(j * GCHUNK, GCHUNK)
            copies.append(pltpu.async_copy(rel_emb.at[idx_r_v.at[s]], rows_r.at[s], sem))
            copies.append(pltpu.async_copy(rel_tr.at[idx_r_v.at[s]], rows_rtr.at[s], sem))
        for c in copies:
            c.wait()
        pltpu.sync_copy(rows_he, ht_out.at[pl.ds(be, E_PER)])
        pltpu.sync_copy(rows_htr, httr_out.at[pl.ds(be, E_PER)])
        pltpu.sync_copy(rows_r, r_out.at[pl.ds(br, R_PER)])
        pltpu.sync_copy(rows_rtr, rtr_out.at[pl.ds(br, R_PER)])

    return _sc_gather


def _loss_body(ht_ref, httr_ref, r_ref, rtr_ref, out_ref):
    r_tr = rtr_ref[...]

    def transfer(e, etr):
        dot = jnp.sum(e * etr, axis=1, keepdims=True)
        e2 = e + dot * r_tr
        n = jnp.sqrt(jnp.sum(e2 * e2, axis=1, keepdims=True))
        return e2 / jnp.maximum(n, 1e-12)

    h = transfer(ht_ref[0:BSEQ, :], httr_ref[0:BSEQ, :])
    t = transfer(ht_ref[BSEQ:E_IDX, :], httr_ref[BSEQ:E_IDX, :])
    a = jnp.abs(h + r_ref[...] - t + 1e-06)
    diff = a[0:BS, :] - a[BS:BSEQ, :]
    rows = jnp.sum(diff, axis=1, keepdims=True)
    out_ref[0, 0] = jnp.sum(jnp.maximum(rows + 1.0, 0.0)) * (1.0 / BS)


_loss_call = pl.pallas_call(
    _loss_body,
    out_specs=pl.BlockSpec(memory_space=pltpu.SMEM),
    out_shape=jax.ShapeDtypeStruct((1, 1), jnp.float32),
)

NCHUNK = 8192
NGRID = -(-ENT_N // NCHUNK)


def _norm_body(a_ref, b_ref, c_ref, d_ref, out_ref):
    i = pl.program_id(0)
    rem = ENT_N - i * NCHUNK

    @pl.when(i == 0)
    def _():
        out_ref[0, 0] = 0.0

    @pl.when(rem >= NCHUNK)
    def _():
        s_ent = jnp.sum(jnp.abs(a_ref[...])) + jnp.sum(jnp.abs(c_ref[...]))
        s_rel = jnp.sum(jnp.abs(b_ref[...])) + jnp.sum(jnp.abs(d_ref[...]))
        out_ref[0, 0] += s_ent * (1.0 / ENT_N) + s_rel * (1.0 / REL_N)

    @pl.when(rem < NCHUNK)
    def _():
        m = jax.lax.broadcasted_iota(jnp.int32, (D, NCHUNK), 1) < rem

        def masked(ref):
            return jnp.sum(jnp.where(m, jnp.abs(ref[...]), 0.0))

        s_ent = masked(a_ref) + masked(c_ref)
        s_rel = masked(b_ref) + masked(d_ref)
        out_ref[0, 0] += s_ent * (1.0 / ENT_N) + s_rel * (1.0 / REL_N)


_norm_call = pl.pallas_call(
    _norm_body,
    grid=(NGRID,),
    in_specs=[pl.BlockSpec((D, NCHUNK), lambda i: (0, i))] * 4,
    out_specs=pl.BlockSpec(memory_space=pltpu.SMEM),
    out_shape=jax.ShapeDtypeStruct((1, 1), jnp.float32),
)


def kernel(input, ent_emb, rel_emb, ent_transfer, rel_transfer):
    idx_e = jnp.concatenate([input[:, 0], input[:, 2]])
    idx_r = input[:, 1]
    ht, httr, r, rtr_g = _sc_gather_fn()(idx_e, idx_r, ent_emb, ent_transfer,
                                         rel_emb, rel_transfer)
    loss = _loss_call(ht, httr, r, rtr_g)
    norm = _norm_call(ent_emb.T, rel_emb.T, ent_transfer.T, rel_transfer.T)
    return loss[0, 0] + norm[0, 0] * REG_C
